# baseline (speedup 1.0000x reference)
import functools

import jax
import jax.numpy as jnp
from jax.experimental import pallas as pl
from jax.experimental.pallas import tpu as pltpu

B, Q, H = 256, 64, 1024
RH = H // 2
K = 1024
N = B * Q
COMMITMENT_WEIGHT = 0.25

BLK_M = 512
GRID = N // BLK_M


def _f32_dot(a, b, dims):
    return jax.lax.dot_general(a, b, dimension_numbers=(dims, ((), ())),
                               preferred_element_type=jnp.float32)


def _body(x_ref, w1_ref, b1_ref, w2_ref, b2_ref, cb_ref, cb2_ref,
          dw1_ref, db1_ref, dw2_ref, db2_ref,
          recon_ref, q_ref, idx_ref, loss_ref):
    i = pl.program_id(0)

    xb = x_ref[...]
    h = jnp.maximum(_f32_dot(xb, w1_ref[...], (((1,), (0,)))) + b1_ref[...], 0.0)
    e = _f32_dot(h, w2_ref[...], (((1,), (0,)))) + b2_ref[...]

    cb = cb_ref[...]
    scores = _f32_dot(e, cb, (((1,), (1,))))
    e2 = jnp.sum(e * e, axis=1, keepdims=True)
    d = e2 - 2.0 * scores + cb2_ref[...]

    dmin = jnp.min(d, axis=1, keepdims=True)
    iota = jax.lax.broadcasted_iota(jnp.int32, d.shape, 1)
    idx = jnp.min(jnp.where(d == dmin, iota, K), axis=1).astype(jnp.int32)
    idx_ref[...] = idx[:, None]

    onehot = (iota == idx[:, None]).astype(jnp.float32)
    q = _f32_dot(onehot, cb, (((1,), (0,))))
    q_ref[...] = q

    diff = q - e
    part = jnp.sum(diff * diff)
    loss_ref[...] = jnp.full((1, 1, 1), 1.0, jnp.float32) * part

    hd = jnp.maximum(_f32_dot(q, dw1_ref[...], (((1,), (0,)))) + db1_ref[...], 0.0)
    recon_ref[...] = _f32_dot(hd, dw2_ref[...], (((1,), (0,)))) + db2_ref[...]


@jax.jit
def _run(x2, enc_w1, enc_b1, enc_w2, enc_b2, codebook, dec_w1, dec_b1, dec_w2, dec_b2):
    full = lambda shape: pl.BlockSpec(shape, lambda i: (0,) * len(shape))
    recon, q, idx, loss = pl.pallas_call(
        _body,
        grid=(GRID,),
        in_specs=[
            pl.BlockSpec((BLK_M, H), lambda i: (i, 0)),
            full((H, H)), full((1, H)), full((H, RH)), full((1, RH)),
            full((K, RH)), full((1, K)),
            full((RH, H)), full((1, H)), full((H, H)), full((1, H)),
        ],
        out_specs=[
            pl.BlockSpec((BLK_M, H), lambda i: (i, 0)),
            pl.BlockSpec((BLK_M, RH), lambda i: (i, 0)),
            pl.BlockSpec((BLK_M, 1), lambda i: (i, 0)),
            pl.BlockSpec((1, 1, 1), lambda i: (i, 0, 0)),
        ],
        out_shape=[
            jax.ShapeDtypeStruct((N, H), jnp.float32),
            jax.ShapeDtypeStruct((N, RH), jnp.float32),
            jax.ShapeDtypeStruct((N, 1), jnp.int32),
            jax.ShapeDtypeStruct((GRID, 1, 1), jnp.float32),
        ],
        compiler_params=pltpu.CompilerParams(
            dimension_semantics=("parallel",)),
    )(x2, enc_w1, enc_b1.reshape(1, H), enc_w2, enc_b2.reshape(1, RH),
      codebook, jnp.sum(codebook * codebook, axis=-1)[None, :],
      dec_w1, dec_b1.reshape(1, H), dec_w2, dec_b2.reshape(1, H))
    return recon, q, idx, loss


def kernel(x, enc_w1, enc_b1, enc_w2, enc_b2, codebook,
           dec_w1, dec_b1, dec_w2, dec_b2):
    x2 = x.reshape(N, H)
    recon, q, idx, loss = _run(x2, enc_w1, enc_b1, enc_w2, enc_b2, codebook,
                               dec_w1, dec_b1, dec_w2, dec_b2)
    vq_loss = COMMITMENT_WEIGHT * (loss.sum() / (N * RH))
    return (recon.reshape(B, Q, H), q.reshape(B, Q, RH),
            idx.reshape(B, Q), x, vq_loss)


# speedup vs baseline: 1.0461x; 1.0461x over previous
import functools

import jax
import jax.numpy as jnp
from jax.experimental import pallas as pl
from jax.experimental.pallas import tpu as pltpu

B, Q, H = 256, 64, 1024
RH = H // 2
K = 1024
N = B * Q
COMMITMENT_WEIGHT = 0.25

BLK_M = 1024
GRID = N // BLK_M


def _f32_dot(a, b, dims):
    return jax.lax.dot_general(a, b, dimension_numbers=(dims, ((), ())),
                               preferred_element_type=jnp.float32)


def _body(x_ref, w1_ref, b1_ref, w2_ref, b2_ref, cb_ref, cb2_ref,
          dw1_ref, db1_ref, dw2_ref, db2_ref,
          recon_ref, q_ref, idx_ref, loss_ref):
    i = pl.program_id(0)

    xb = x_ref[...]
    h = jnp.maximum(_f32_dot(xb, w1_ref[...], (((1,), (0,)))) + b1_ref[...], 0.0)
    e = _f32_dot(h, w2_ref[...], (((1,), (0,)))) + b2_ref[...]

    cb = cb_ref[...]
    scores = _f32_dot(e, cb, (((1,), (1,))))
    e2 = jnp.sum(e * e, axis=1, keepdims=True)
    d = e2 - 2.0 * scores + cb2_ref[...]

    dmin = jnp.min(d, axis=1, keepdims=True)
    iota = jax.lax.broadcasted_iota(jnp.int32, d.shape, 1)
    idx = jnp.min(jnp.where(d == dmin, iota, K), axis=1).astype(jnp.int32)
    idx_ref[...] = idx[:, None]

    onehot = (iota == idx[:, None]).astype(jnp.float32)
    q = _f32_dot(onehot, cb, (((1,), (0,))))
    q_ref[...] = q

    diff = q - e
    part = jnp.sum(diff * diff)
    loss_ref[...] = jnp.full((1, 1, 1), 1.0, jnp.float32) * part

    hd = jnp.maximum(_f32_dot(q, dw1_ref[...], (((1,), (0,)))) + db1_ref[...], 0.0)
    recon_ref[...] = _f32_dot(hd, dw2_ref[...], (((1,), (0,)))) + db2_ref[...]


@jax.jit
def _run(x2, enc_w1, enc_b1, enc_w2, enc_b2, codebook, dec_w1, dec_b1, dec_w2, dec_b2):
    full = lambda shape: pl.BlockSpec(shape, lambda i: (0,) * len(shape))
    recon, q, idx, loss = pl.pallas_call(
        _body,
        grid=(GRID,),
        in_specs=[
            pl.BlockSpec((BLK_M, H), lambda i: (i, 0)),
            full((H, H)), full((1, H)), full((H, RH)), full((1, RH)),
            full((K, RH)), full((1, K)),
            full((RH, H)), full((1, H)), full((H, H)), full((1, H)),
        ],
        out_specs=[
            pl.BlockSpec((BLK_M, H), lambda i: (i, 0)),
            pl.BlockSpec((BLK_M, RH), lambda i: (i, 0)),
            pl.BlockSpec((BLK_M, 1), lambda i: (i, 0)),
            pl.BlockSpec((1, 1, 1), lambda i: (i, 0, 0)),
        ],
        out_shape=[
            jax.ShapeDtypeStruct((N, H), jnp.float32),
            jax.ShapeDtypeStruct((N, RH), jnp.float32),
            jax.ShapeDtypeStruct((N, 1), jnp.int32),
            jax.ShapeDtypeStruct((GRID, 1, 1), jnp.float32),
        ],
        compiler_params=pltpu.CompilerParams(
            dimension_semantics=("parallel",)),
    )(x2, enc_w1, enc_b1.reshape(1, H), enc_w2, enc_b2.reshape(1, RH),
      codebook, jnp.sum(codebook * codebook, axis=-1)[None, :],
      dec_w1, dec_b1.reshape(1, H), dec_w2, dec_b2.reshape(1, H))
    return recon, q, idx, loss


def kernel(x, enc_w1, enc_b1, enc_w2, enc_b2, codebook,
           dec_w1, dec_b1, dec_w2, dec_b2):
    x2 = x.reshape(N, H)
    recon, q, idx, loss = _run(x2, enc_w1, enc_b1, enc_w2, enc_b2, codebook,
                               dec_w1, dec_b1, dec_w2, dec_b2)
    vq_loss = COMMITMENT_WEIGHT * (loss.sum() / (N * RH))
    return (recon.reshape(B, Q, H), q.reshape(B, Q, RH),
            idx.reshape(B, Q), x, vq_loss)
